# baseline (speedup 1.0000x reference)
import jax
import jax.numpy as jnp
from jax.experimental import pallas as pl
from jax.experimental.pallas import tpu as pltpu

B = 64
K = 512
P = 196
F = 384
D = P * F
N_PIX = 64 * 3 * 224 * 224

IMG_SLABS = 192
MSE_STEPS = 12
IMG_SBLK = IMG_SLABS // MSE_STEPS

SOM_STEPS = 14
PBLK = P // SOM_STEPS
DBLK = PBLK * F


def _mse_body(a_ref, b_ref, out_ref, acc):
    i = pl.program_id(0)
    d = a_ref[...] - b_ref[...]
    part = jnp.sum(d * d)

    @pl.when(i == 0)
    def _init():
        acc[0] = part

    @pl.when(i > 0)
    def _accum():
        acc[0] += part

    @pl.when(i == MSE_STEPS - 1)
    def _fin():
        out_ref[0] = acc[0] * (1.0 / N_PIX)


def _som_body(x_hbm, y_ref, gc_ref, sig_ref, lam_ref, lnn_ref,
              lt_ref, ln_ref, ls_ref,
              x_vmem, g_acc, sx_acc, sy_acc, x_sem):
    i = pl.program_id(0)

    @pl.when(i == 0)
    def _fetch_x():
        cp = pltpu.make_async_copy(x_hbm, x_vmem, x_sem)
        cp.start()
        cp.wait()

    p0 = 1 + i * PBLK
    xb = jnp.concatenate([x_vmem[:, p0 + j, :] for j in range(PBLK)],
                         axis=1)
    yb = y_ref[...]
    g = jax.lax.dot_general(xb, yb, (((1,), (1,)), ((), ())),
                            preferred_element_type=jnp.float32)
    sxp = jnp.sum(xb * xb, axis=1, keepdims=True)
    syp = jnp.sum(yb * yb, axis=1, keepdims=True)

    @pl.when(i == 0)
    def _init():
        g_acc[...] = g
        sx_acc[...] = sxp
        sy_acc[...] = syp

    @pl.when(i > 0)
    def _accum():
        g_acc[...] += g
        sx_acc[...] += sxp
        sy_acc[...] += syp

    @pl.when(i == SOM_STEPS - 1)
    def _epilogue():
        eps = 1e-8
        hi = jax.lax.Precision.HIGHEST
        iota_r = jax.lax.broadcasted_iota(jnp.int32, (K, K), 0)
        iota_c = jax.lax.broadcasted_iota(jnp.int32, (K, K), 1)
        eye = (iota_r == iota_c).astype(jnp.float32)
        sy_row = jax.lax.dot_general(sy_acc[...], eye, (((0,), (0,)), ((), ())),
                                     preferred_element_type=jnp.float32,
                                     precision=hi)
        nx = jnp.sqrt(sx_acc[...]) + eps
        ny = jnp.sqrt(sy_row) + eps
        dists = 1.0 - g_acc[...] / (nx * ny)
        m = jnp.min(dists, axis=1, keepdims=True)
        iota = jax.lax.broadcasted_iota(jnp.int32, (B, K), 1)
        idx = jnp.min(jnp.where(dists == m, iota, K),
                      axis=1, keepdims=True)
        onehot = (iota == idx).astype(jnp.float32)
        gc = gc_ref[...]
        bmu = jax.lax.dot_general(onehot, gc, (((1,), (0,)), ((), ())),
                                  preferred_element_type=jnp.float32,
                                  precision=hi)
        ca2 = jnp.sum(bmu * bmu, axis=1, keepdims=True)
        cc2 = jax.lax.dot_general(jnp.ones((1, 2), jnp.float32), gc * gc,
                                  (((1,), (1,)), ((), ())),
                                  preferred_element_type=jnp.float32,
                                  precision=hi)
        cross = jax.lax.dot_general(bmu, gc, (((1,), (1,)), ((), ())),
                                    preferred_element_type=jnp.float32,
                                    precision=hi)
        dist_grid = jnp.maximum(ca2 + cc2 - 2.0 * cross, 0.0)
        sig = sig_ref[0]
        neigh = jnp.exp(-dist_grid / (2.0 * sig * sig))
        lsom = jnp.sum(neigh * dists) * (1.0 / B)
        lnn = lnn_ref[0]
        ls_ref[0] = lsom
        ln_ref[0] = lnn
        lt_ref[0] = lam_ref[0] * lsom + lnn


def kernel(original_img, reconstructed, latent_vectors, som_weights,
           grid_coords, sigma, current_lamda):
    a = original_img.reshape(IMG_SLABS, 224, 224)
    b = reconstructed.reshape(IMG_SLABS, 224, 224)
    sig = sigma.reshape(1).astype(jnp.float32)
    lam = current_lamda.reshape(1).astype(jnp.float32)

    smem = pltpu.SMEM
    lnn = pl.pallas_call(
        _mse_body,
        grid=(MSE_STEPS,),
        in_specs=[
            pl.BlockSpec((IMG_SBLK, 224, 224), lambda i: (i, 0, 0)),
            pl.BlockSpec((IMG_SBLK, 224, 224), lambda i: (i, 0, 0)),
        ],
        out_specs=pl.BlockSpec(memory_space=smem),
        out_shape=jax.ShapeDtypeStruct((1,), jnp.float32),
        scratch_shapes=[pltpu.SMEM((1,), jnp.float32)],
    )(a, b)

    lt, ln, ls = pl.pallas_call(
        _som_body,
        grid=(SOM_STEPS,),
        in_specs=[
            pl.BlockSpec(memory_space=pl.ANY),
            pl.BlockSpec((K, DBLK), lambda i: (0, i)),
            pl.BlockSpec((K, 2), lambda i: (0, 0)),
            pl.BlockSpec(memory_space=smem),
            pl.BlockSpec(memory_space=smem),
            pl.BlockSpec(memory_space=smem),
        ],
        out_specs=[
            pl.BlockSpec(memory_space=smem),
            pl.BlockSpec(memory_space=smem),
            pl.BlockSpec(memory_space=smem),
        ],
        out_shape=[jax.ShapeDtypeStruct((1,), jnp.float32)] * 3,
        scratch_shapes=[
            pltpu.VMEM((B, 197, F), jnp.float32),
            pltpu.VMEM((B, K), jnp.float32),
            pltpu.VMEM((B, 1), jnp.float32),
            pltpu.VMEM((K, 1), jnp.float32),
            pltpu.SemaphoreType.DMA,
        ],
    )(latent_vectors, som_weights, grid_coords, sig, lam, lnn)
    return (lt[0], ln[0], ls[0])


# speedup vs baseline: 1.3104x; 1.3104x over previous
import jax
import jax.numpy as jnp
from jax.experimental import pallas as pl
from jax.experimental.pallas import tpu as pltpu

B = 64
K = 512
P = 196
F = 384
D = P * F
N_PIX = 64 * 3 * 224 * 224

IMG_SLABS = 192
MSE_STEPS = 12
IMG_SBLK = IMG_SLABS // MSE_STEPS

SOM_STEPS = 14
PBLK = P // SOM_STEPS
DBLK = PBLK * F


def _mse_body(a_ref, b_ref, out_ref, acc):
    i = pl.program_id(0)
    d = a_ref[...] - b_ref[...]
    part = jnp.sum(d * d)

    @pl.when(i == 0)
    def _init():
        acc[0] = part

    @pl.when(i > 0)
    def _accum():
        acc[0] += part

    @pl.when(i == MSE_STEPS - 1)
    def _fin():
        out_ref[0] = acc[0] * (1.0 / N_PIX)


def _som_body(*refs):
    (x_refs, (y_ref, gc_ref, sig_ref, lam_ref, lnn_ref,
              lt_ref, ln_ref, ls_ref, g_acc, sx_acc, sy_acc)) = \
        refs[:PBLK], refs[PBLK:]
    i = pl.program_id(0)

    xb = jnp.concatenate([r[...] for r in x_refs], axis=1)
    yb = y_ref[...]
    g = jax.lax.dot_general(xb, yb, (((1,), (1,)), ((), ())),
                            preferred_element_type=jnp.float32)
    sxp = jnp.sum(xb * xb, axis=1, keepdims=True)
    syp = jnp.sum(yb * yb, axis=1, keepdims=True)

    @pl.when(i == 0)
    def _init():
        g_acc[...] = g
        sx_acc[...] = sxp
        sy_acc[...] = syp

    @pl.when(i > 0)
    def _accum():
        g_acc[...] += g
        sx_acc[...] += sxp
        sy_acc[...] += syp

    @pl.when(i == SOM_STEPS - 1)
    def _epilogue():
        eps = 1e-8
        hi = jax.lax.Precision.HIGHEST
        iota_r = jax.lax.broadcasted_iota(jnp.int32, (K, K), 0)
        iota_c = jax.lax.broadcasted_iota(jnp.int32, (K, K), 1)
        eye = (iota_r == iota_c).astype(jnp.float32)
        sy_row = jax.lax.dot_general(sy_acc[...], eye, (((0,), (0,)), ((), ())),
                                     preferred_element_type=jnp.float32,
                                     precision=hi)
        nx = jnp.sqrt(sx_acc[...]) + eps
        ny = jnp.sqrt(sy_row) + eps
        dists = 1.0 - g_acc[...] / (nx * ny)
        m = jnp.min(dists, axis=1, keepdims=True)
        iota = jax.lax.broadcasted_iota(jnp.int32, (B, K), 1)
        idx = jnp.min(jnp.where(dists == m, iota, K),
                      axis=1, keepdims=True)
        onehot = (iota == idx).astype(jnp.float32)
        gc = gc_ref[...]
        bmu = jax.lax.dot_general(onehot, gc, (((1,), (0,)), ((), ())),
                                  preferred_element_type=jnp.float32,
                                  precision=hi)
        ca2 = jnp.sum(bmu * bmu, axis=1, keepdims=True)
        cc2 = jax.lax.dot_general(jnp.ones((1, 2), jnp.float32), gc * gc,
                                  (((1,), (1,)), ((), ())),
                                  preferred_element_type=jnp.float32,
                                  precision=hi)
        cross = jax.lax.dot_general(bmu, gc, (((1,), (1,)), ((), ())),
                                    preferred_element_type=jnp.float32,
                                    precision=hi)
        dist_grid = jnp.maximum(ca2 + cc2 - 2.0 * cross, 0.0)
        sig = sig_ref[0]
        neigh = jnp.exp(-dist_grid / (2.0 * sig * sig))
        lsom = jnp.sum(neigh * dists) * (1.0 / B)
        lnn = lnn_ref[0]
        ls_ref[0] = lsom
        ln_ref[0] = lnn
        lt_ref[0] = lam_ref[0] * lsom + lnn


def kernel(original_img, reconstructed, latent_vectors, som_weights,
           grid_coords, sigma, current_lamda):
    a = original_img.reshape(IMG_SLABS, 224, 224)
    b = reconstructed.reshape(IMG_SLABS, 224, 224)
    xt = jnp.transpose(latent_vectors, (1, 0, 2))
    sig = sigma.reshape(1).astype(jnp.float32)
    lam = current_lamda.reshape(1).astype(jnp.float32)

    smem = pltpu.SMEM
    lnn = pl.pallas_call(
        _mse_body,
        grid=(MSE_STEPS,),
        in_specs=[
            pl.BlockSpec((IMG_SBLK, 224, 224), lambda i: (i, 0, 0)),
            pl.BlockSpec((IMG_SBLK, 224, 224), lambda i: (i, 0, 0)),
        ],
        out_specs=pl.BlockSpec(memory_space=smem),
        out_shape=jax.ShapeDtypeStruct((1,), jnp.float32),
        scratch_shapes=[pltpu.SMEM((1,), jnp.float32)],
    )(a, b)

    x_specs = [
        pl.BlockSpec((None, B, F), lambda i, j=j: (1 + PBLK * i + j, 0, 0))
        for j in range(PBLK)
    ]
    lt, ln, ls = pl.pallas_call(
        _som_body,
        grid=(SOM_STEPS,),
        in_specs=x_specs + [
            pl.BlockSpec((K, DBLK), lambda i: (0, i)),
            pl.BlockSpec((K, 2), lambda i: (0, 0)),
            pl.BlockSpec(memory_space=smem),
            pl.BlockSpec(memory_space=smem),
            pl.BlockSpec(memory_space=smem),
        ],
        out_specs=[
            pl.BlockSpec(memory_space=smem),
            pl.BlockSpec(memory_space=smem),
            pl.BlockSpec(memory_space=smem),
        ],
        out_shape=[jax.ShapeDtypeStruct((1,), jnp.float32)] * 3,
        scratch_shapes=[
            pltpu.VMEM((B, K), jnp.float32),
            pltpu.VMEM((B, 1), jnp.float32),
            pltpu.VMEM((K, 1), jnp.float32),
        ],
    )(*([xt] * PBLK), som_weights, grid_coords, sig, lam, lnn)
    return (lt[0], ln[0], ls[0])


# speedup vs baseline: 1.3121x; 1.0013x over previous
import jax
import jax.numpy as jnp
from jax.experimental import pallas as pl
from jax.experimental.pallas import tpu as pltpu

B = 64
K = 512
P = 196
F = 384
D = P * F
N_PIX = 64 * 3 * 224 * 224

IMG_SLABS = 192
MSE_STEPS = 12
IMG_SBLK = IMG_SLABS // MSE_STEPS

SOM_STEPS = 14
PBLK = P // SOM_STEPS
DBLK = PBLK * F


def _mse_body(a_ref, b_ref, out_ref, acc):
    i = pl.program_id(0)
    d = a_ref[...] - b_ref[...]
    part = jnp.sum(d * d)

    @pl.when(i == 0)
    def _init():
        acc[0] = part

    @pl.when(i > 0)
    def _accum():
        acc[0] += part

    @pl.when(i == MSE_STEPS - 1)
    def _fin():
        out_ref[0] = acc[0] * (1.0 / N_PIX)


def _som_body(*refs):
    (x_refs, (y0_ref, y1_ref, gc_ref, sig_ref, lam_ref, lnn_ref,
              lt_ref, ln_ref, ls_ref, g_acc, sx_acc, sy_acc)) = \
        refs[:PBLK], refs[PBLK:]
    i = pl.program_id(0)

    xb = jnp.concatenate([r[...] for r in x_refs], axis=1)
    yb0 = y0_ref[...]
    yb1 = y1_ref[...]
    g = jnp.concatenate(
        [jax.lax.dot_general(xb, yh, (((1,), (1,)), ((), ())),
                             preferred_element_type=jnp.float32)
         for yh in (yb0, yb1)], axis=1)
    sxp = jnp.sum(xb * xb, axis=1, keepdims=True)
    syp = jnp.concatenate(
        [jnp.sum(yh * yh, axis=1, keepdims=True) for yh in (yb0, yb1)],
        axis=0)

    @pl.when(i == 0)
    def _init():
        g_acc[...] = g
        sx_acc[...] = sxp
        sy_acc[...] = syp

    @pl.when(i > 0)
    def _accum():
        g_acc[...] += g
        sx_acc[...] += sxp
        sy_acc[...] += syp

    @pl.when(i == SOM_STEPS - 1)
    def _epilogue():
        eps = 1e-8
        hi = jax.lax.Precision.HIGHEST
        iota_r = jax.lax.broadcasted_iota(jnp.int32, (K, K), 0)
        iota_c = jax.lax.broadcasted_iota(jnp.int32, (K, K), 1)
        eye = (iota_r == iota_c).astype(jnp.float32)
        sy_row = jax.lax.dot_general(sy_acc[...], eye, (((0,), (0,)), ((), ())),
                                     preferred_element_type=jnp.float32,
                                     precision=hi)
        nx = jnp.sqrt(sx_acc[...]) + eps
        ny = jnp.sqrt(sy_row) + eps
        dists = 1.0 - g_acc[...] / (nx * ny)
        m = jnp.min(dists, axis=1, keepdims=True)
        iota = jax.lax.broadcasted_iota(jnp.int32, (B, K), 1)
        idx = jnp.min(jnp.where(dists == m, iota, K),
                      axis=1, keepdims=True)
        onehot = (iota == idx).astype(jnp.float32)
        gc = gc_ref[...]
        bmu = jax.lax.dot_general(onehot, gc, (((1,), (0,)), ((), ())),
                                  preferred_element_type=jnp.float32,
                                  precision=hi)
        ca2 = jnp.sum(bmu * bmu, axis=1, keepdims=True)
        cc2 = jax.lax.dot_general(jnp.ones((1, 2), jnp.float32), gc * gc,
                                  (((1,), (1,)), ((), ())),
                                  preferred_element_type=jnp.float32,
                                  precision=hi)
        cross = jax.lax.dot_general(bmu, gc, (((1,), (1,)), ((), ())),
                                    preferred_element_type=jnp.float32,
                                    precision=hi)
        dist_grid = jnp.maximum(ca2 + cc2 - 2.0 * cross, 0.0)
        sig = sig_ref[0]
        neigh = jnp.exp(-dist_grid / (2.0 * sig * sig))
        lsom = jnp.sum(neigh * dists) * (1.0 / B)
        lnn = lnn_ref[0]
        ls_ref[0] = lsom
        ln_ref[0] = lnn
        lt_ref[0] = lam_ref[0] * lsom + lnn


def kernel(original_img, reconstructed, latent_vectors, som_weights,
           grid_coords, sigma, current_lamda):
    a = original_img.reshape(IMG_SLABS, 224, 224)
    b = reconstructed.reshape(IMG_SLABS, 224, 224)
    xt = jnp.transpose(latent_vectors, (1, 0, 2))
    sig = sigma.reshape(1).astype(jnp.float32)
    lam = current_lamda.reshape(1).astype(jnp.float32)

    smem = pltpu.SMEM
    lnn = pl.pallas_call(
        _mse_body,
        grid=(MSE_STEPS,),
        in_specs=[
            pl.BlockSpec((IMG_SBLK, 224, 224), lambda i: (i, 0, 0)),
            pl.BlockSpec((IMG_SBLK, 224, 224), lambda i: (i, 0, 0)),
        ],
        out_specs=pl.BlockSpec(memory_space=smem),
        out_shape=jax.ShapeDtypeStruct((1,), jnp.float32),
        scratch_shapes=[pltpu.SMEM((1,), jnp.float32)],
    )(a, b)

    x_specs = [
        pl.BlockSpec((None, B, F), lambda i, j=j: (1 + PBLK * i + j, 0, 0))
        for j in range(PBLK)
    ]
    lt, ln, ls = pl.pallas_call(
        _som_body,
        grid=(SOM_STEPS,),
        in_specs=x_specs + [
            pl.BlockSpec((K // 2, DBLK), lambda i: (0, i)),
            pl.BlockSpec((K // 2, DBLK), lambda i: (1, i)),
            pl.BlockSpec((K, 2), lambda i: (0, 0)),
            pl.BlockSpec(memory_space=smem),
            pl.BlockSpec(memory_space=smem),
            pl.BlockSpec(memory_space=smem),
        ],
        out_specs=[
            pl.BlockSpec(memory_space=smem),
            pl.BlockSpec(memory_space=smem),
            pl.BlockSpec(memory_space=smem),
        ],
        out_shape=[jax.ShapeDtypeStruct((1,), jnp.float32)] * 3,
        scratch_shapes=[
            pltpu.VMEM((B, K), jnp.float32),
            pltpu.VMEM((B, 1), jnp.float32),
            pltpu.VMEM((K, 1), jnp.float32),
        ],
    )(*([xt] * PBLK), som_weights, som_weights, grid_coords, sig, lam, lnn)
    return (lt[0], ln[0], ls[0])
